# baseline (speedup 1.0000x reference)
import functools

import jax
import jax.numpy as jnp
from jax import lax
from jax.experimental import pallas as pl
from jax.experimental.pallas import tpu as pltpu
from jax.experimental.pallas import tpu_sc as plsc

_E = 8
_H = 1024
_I = 512
_T = 2048
_K = 2
_P = _T * _K

_B = 128
_S = _P + _E * _B
_NB = _S // _B

_Q = 4
_QD = _H // _Q
_W = 128


@functools.cache
def _vector_mesh():
    return plsc.VectorSubcoreMesh(core_axis_name="c", subcore_axis_name="s",
                                  num_cores=2, num_subcores=16)


def _sc_gather_body(table_hbm, idx_hbm, out_hbm):
    def body(i_vmem, o_vmem):
        pltpu.sync_copy(table_hbm.at[i_vmem.at[0]], o_vmem)

    pltpu.emit_pipeline(
        body,
        grid=(_S * _Q // _W,),
        in_specs=[pl.BlockSpec((1, _W), lambda i: (0, i))],
        out_specs=[pl.BlockSpec((_W, _QD), lambda i: (i, 0))],
        core_axis_name=("c", "s"),
        dimension_semantics=(pltpu.PARALLEL,),
    )(idx_hbm, out_hbm)


def _sc_gather(table_q, src_q):
    out = pl.kernel(
        _sc_gather_body,
        out_type=jax.ShapeDtypeStruct((_S * _Q, _QD), jnp.float32),
        mesh=_vector_mesh(),
    )(table_q, src_q.reshape(1, _S * _Q))
    return out.reshape(_S, _H)


def _sc_invert_body(dest_hbm, w_hbm, src_out, roww_out,
                    dest_v, w_v, srcbuf, rowwbuf):
    wid = lax.axis_index("s") * 2 + lax.axis_index("c")

    @pl.when(wid == 0)
    def _():
        pltpu.sync_copy(dest_hbm, dest_v)
        pltpu.sync_copy(w_hbm, w_v)
        zero_i = jnp.zeros((16,), jnp.int32)
        zero_f = jnp.zeros((16,), jnp.float32)

        @pl.loop(0, _S, step=16)
        def _zero(j):
            srcbuf[pl.ds(j, 16)] = zero_i
            rowwbuf[pl.ds(j, 16)] = zero_f

        @pl.loop(0, _P, step=16)
        def _scatter(p):
            d = dest_v[pl.ds(p, 16)]
            tok = lax.shift_right_logical(p + lax.iota(jnp.int32, 16), 1)
            plsc.store_scatter(srcbuf, [d], tok)
            plsc.store_scatter(rowwbuf, [d], w_v[pl.ds(p, 16)])

        pltpu.sync_copy(srcbuf, src_out)
        pltpu.sync_copy(rowwbuf, roww_out)


def _sc_invert(dest, w_flat):
    import dataclasses
    cp = pltpu.CompilerParams()
    if "needs_layout_passes" in pltpu.CompilerParams.__dataclass_fields__:
        cp = dataclasses.replace(cp, needs_layout_passes=False)
    return pl.kernel(
        _sc_invert_body,
        compiler_params=cp,
        out_type=(jax.ShapeDtypeStruct((_S,), jnp.int32),
                  jax.ShapeDtypeStruct((_S,), jnp.float32)),
        mesh=_vector_mesh(),
        scratch_types=[
            pltpu.VMEM((_P,), jnp.int32),
            pltpu.VMEM((_P,), jnp.float32),
            pltpu.VMEM((_S,), jnp.int32),
            pltpu.VMEM((_S,), jnp.float32),
        ],
    )(dest, w_flat)


def _ffn_kernel(be_ref, xs_ref, w_ref, gup_ref, down_ref, ys_ref):
    e = be_ref[pl.program_id(0)]
    x = xs_ref[...].astype(jnp.bfloat16)
    gu = lax.dot_general(
        x, gup_ref[e],
        (((1,), (1,)), ((), ())),
        preferred_element_type=jnp.float32,
    )
    gate = gu[:, :_I]
    up = gu[:, _I:]
    h = (gate * jax.nn.sigmoid(gate) * up).astype(jnp.bfloat16)
    y = lax.dot_general(
        h, down_ref[e],
        (((1,), (1,)), ((), ())),
        preferred_element_type=jnp.float32,
    )
    ys_ref[...] = y * w_ref[...]


def _tc_ffn(block_expert, xs, row_w, gup16, down16):
    grid_spec = pltpu.PrefetchScalarGridSpec(
        num_scalar_prefetch=1,
        grid=(_NB,),
        in_specs=[
            pl.BlockSpec((_B, _H), lambda b, be: (b, 0)),
            pl.BlockSpec((_B, 1), lambda b, be: (b, 0)),
            pl.BlockSpec((_E, 2 * _I, _H), lambda b, be: (0, 0, 0)),
            pl.BlockSpec((_E, _H, _I), lambda b, be: (0, 0, 0)),
        ],
        out_specs=pl.BlockSpec((_B, _H), lambda b, be: (b, 0)),
    )
    return pl.pallas_call(
        _ffn_kernel,
        grid_spec=grid_spec,
        out_shape=jax.ShapeDtypeStruct((_S, _H), jnp.float32),
        compiler_params=pltpu.CompilerParams(
            vmem_limit_bytes=100 * 1024 * 1024,
        ),
    )(block_expert, xs, row_w, gup16, down16)


def _sc_combine_body(ys_hbm, p0_hbm, p1_hbm, out_hbm, buf1, sem):
    def body(i0_vmem, i1_vmem, o_vmem):
        cp1 = pltpu.async_copy(ys_hbm.at[i1_vmem.at[0]], buf1, sem)
        pltpu.sync_copy(ys_hbm.at[i0_vmem.at[0]], o_vmem)
        cp1.wait()

        @pl.loop(0, _W)
        def _rows(r):
            for u in range(0, _QD, 16):
                slc = (pl.ds(r, 1), pl.ds(u, 16))
                o_vmem.at[slc][...] = o_vmem.at[slc][...] + buf1.at[slc][...]

    pltpu.emit_pipeline(
        body,
        grid=(_T * _Q // _W,),
        in_specs=[pl.BlockSpec((1, _W), lambda i: (0, i)),
                  pl.BlockSpec((1, _W), lambda i: (0, i))],
        out_specs=[pl.BlockSpec((_W, _QD), lambda i: (i, 0))],
        core_axis_name=("c", "s"),
        dimension_semantics=(pltpu.PARALLEL,),
    )(p0_hbm, p1_hbm, out_hbm)


def _sc_combine(ys_q, p0_q, p1_q):
    out = pl.kernel(
        _sc_combine_body,
        out_type=jax.ShapeDtypeStruct((_T * _Q, _QD), jnp.float32),
        mesh=_vector_mesh(),
        scratch_types=[
            pltpu.VMEM((_W, _QD), jnp.float32),
            pltpu.SemaphoreType.DMA,
        ],
    )(ys_q, p0_q.reshape(1, _T * _Q), p1_q.reshape(1, _T * _Q))
    return out.reshape(_T, _H)


def kernel(hidden_states, top_k_index, top_k_weights, gate_up_proj, down_proj):
    gup16 = gate_up_proj.astype(jnp.bfloat16)
    down16 = down_proj.astype(jnp.bfloat16)

    e_flat = top_k_index.astype(jnp.int32).reshape(-1)
    w_flat = top_k_weights.reshape(-1)
    onehot = (e_flat[:, None] == jnp.arange(_E, dtype=jnp.int32)).astype(jnp.int32)
    csum = jnp.cumsum(onehot, axis=0)
    counts = csum[-1]
    rank = jnp.take_along_axis(csum, e_flat[:, None], axis=1)[:, 0] - 1
    pc = ((counts + _B - 1) // _B) * _B
    off = jnp.concatenate(
        [jnp.zeros((1,), jnp.int32), jnp.cumsum(pc)[:-1].astype(jnp.int32)])
    dest = off[e_flat] + rank
    src_token, row_w1 = _sc_invert(dest, w_flat)
    row_w = row_w1[:, None]
    cumblk = jnp.cumsum(pc // _B)
    block_expert = jnp.minimum(
        jnp.searchsorted(cumblk, jnp.arange(_NB, dtype=jnp.int32), side="right"),
        _E - 1).astype(jnp.int32)
    pos = dest.reshape(_T, _K)

    quarters = jnp.arange(_Q, dtype=jnp.int32)
    src_q = (src_token[:, None] * _Q + quarters).reshape(-1)
    p0_q = (pos[:, 0:1] * _Q + quarters).reshape(-1)
    p1_q = (pos[:, 1:2] * _Q + quarters).reshape(-1)

    table_q = hidden_states.reshape(_T * _Q, _QD)
    xs = _sc_gather(table_q, src_q)
    ys = _tc_ffn(block_expert, xs, row_w, gup16, down16)
    ys_q = ys.reshape(_S * _Q, _QD)
    return _sc_combine(ys_q, p0_q, p1_q)


# speedup vs baseline: 3.0497x; 3.0497x over previous
import functools

import jax
import jax.numpy as jnp
from jax.experimental import pallas as pl
from jax.experimental.pallas import tpu as pltpu

_E = 8
_H = 1024
_I = 512
_T = 2048
_K = 2
_TB = 512


def _dense_moe_kernel(idx_ref, w_ref, x_ref, gup_ref, down_ref, out_ref):
    x = x_ref[...]
    idx = idx_ref[...]
    w = w_ref[...]
    acc = jnp.zeros(out_ref.shape, jnp.float32)
    for e in range(_E):
        gu = jax.lax.dot_general(
            x, gup_ref[e],
            (((1,), (1,)), ((), ())),
            preferred_element_type=jnp.float32,
        )
        gate = gu[:, :_I]
        up = gu[:, _I:]
        h = (gate * jax.nn.sigmoid(gate) * up).astype(jnp.bfloat16)
        y = jax.lax.dot_general(
            h, down_ref[e],
            (((1,), (1,)), ((), ())),
            preferred_element_type=jnp.float32,
        )
        c = jnp.sum(jnp.where(idx == e, w, 0.0), axis=1, keepdims=True)
        acc = acc + y * c
    out_ref[...] = acc


def kernel(hidden_states, top_k_index, top_k_weights, gate_up_proj, down_proj):
    x16 = hidden_states.astype(jnp.bfloat16)
    gup16 = gate_up_proj.astype(jnp.bfloat16)
    down16 = down_proj.astype(jnp.bfloat16)
    idx32 = top_k_index.astype(jnp.int32)

    return pl.pallas_call(
        _dense_moe_kernel,
        grid=(_T // _TB,),
        in_specs=[
            pl.BlockSpec((_TB, _K), lambda i: (i, 0)),
            pl.BlockSpec((_TB, _K), lambda i: (i, 0)),
            pl.BlockSpec((_TB, _H), lambda i: (i, 0)),
            pl.BlockSpec((_E, 2 * _I, _H), lambda i: (0, 0, 0)),
            pl.BlockSpec((_E, _H, _I), lambda i: (0, 0, 0)),
        ],
        out_specs=pl.BlockSpec((_TB, _H), lambda i: (i, 0)),
        out_shape=jax.ShapeDtypeStruct((_T, _H), jnp.float32),
        compiler_params=pltpu.CompilerParams(
            vmem_limit_bytes=100 * 1024 * 1024,
        ),
    )(idx32, top_k_weights, x16, gup16, down16)


# speedup vs baseline: 3.2860x; 1.0775x over previous
import functools

import jax
import jax.numpy as jnp
from jax.experimental import pallas as pl
from jax.experimental.pallas import tpu as pltpu

_E = 8
_H = 1024
_I = 512
_T = 2048
_K = 2
_TB = 1024


def _dense_moe_kernel(idx_ref, w_ref, x_ref, gup_ref, down_ref, out_ref):
    x = x_ref[...].astype(jnp.bfloat16)
    idx = idx_ref[...]
    w = w_ref[...]
    acc = jnp.zeros(out_ref.shape, jnp.float32)
    for e in range(_E):
        gu = jax.lax.dot_general(
            x, gup_ref[e],
            (((1,), (1,)), ((), ())),
            preferred_element_type=jnp.float32,
        )
        gate = gu[:, :_I]
        up = gu[:, _I:]
        h = (gate * jax.nn.sigmoid(gate) * up).astype(jnp.bfloat16)
        y = jax.lax.dot_general(
            h, down_ref[e],
            (((1,), (1,)), ((), ())),
            preferred_element_type=jnp.float32,
        )
        c = jnp.sum(jnp.where(idx == e, w, 0.0), axis=1, keepdims=True)
        acc = acc + y * c
    out_ref[...] = acc


def kernel(hidden_states, top_k_index, top_k_weights, gate_up_proj, down_proj):
    gup16 = gate_up_proj.astype(jnp.bfloat16)
    down16 = down_proj.astype(jnp.bfloat16)
    idx32 = top_k_index.astype(jnp.int32)

    return pl.pallas_call(
        _dense_moe_kernel,
        grid=(_T // _TB,),
        in_specs=[
            pl.BlockSpec((_TB, _K), lambda i: (i, 0)),
            pl.BlockSpec((_TB, _K), lambda i: (i, 0)),
            pl.BlockSpec((_TB, _H), lambda i: (i, 0)),
            pl.BlockSpec((_E, 2 * _I, _H), lambda i: (0, 0, 0)),
            pl.BlockSpec((_E, _H, _I), lambda i: (0, 0, 0)),
        ],
        out_specs=pl.BlockSpec((_TB, _H), lambda i: (i, 0)),
        out_shape=jax.ShapeDtypeStruct((_T, _H), jnp.float32),
        compiler_params=pltpu.CompilerParams(
            vmem_limit_bytes=100 * 1024 * 1024,
        ),
    )(idx32, top_k_weights, hidden_states, gup16, down16)
